# baseline (speedup 1.0000x reference)
import functools

import jax
import jax.numpy as jnp
from jax import lax
from jax.experimental import pallas as pl
from jax.experimental.pallas import tpu as pltpu
from jax.experimental.pallas import tpu_sc as plsc

N = 50000
NP = 51200
E = 800000
ER = E // 128
G = 256
GP = 272
BN_EPS = 1e-5
BLK = 1024
NBLK = NP // BLK

_mesh = plsc.VectorSubcoreMesh(core_axis_name="c", subcore_axis_name="s")


def _make_msg_body(fw, l1):
    nh = fw // 16
    rps = 1 if l1 else 2
    S = 198
    HEX = S // 6

    def body(table, edata, eattr, wpk, out, ed_v, ea_v, rows_v, w_v, aggr,
             sem_i, sem_g, sem_s):
        c = lax.axis_index("c")
        s = lax.axis_index("s")
        if l1:
            pltpu.sync_copy(wpk, w_v)
            base = (s * 2 + c) * S
            cN = None
        else:
            pltpu.sync_copy(wpk.at[c], w_v)
            base = s * S * 2
            cN = c * NP
        ws = [[w_v[r, pl.ds(h * 16, 16)] for h in range(nh)] for r in range(3)]
        z16 = jnp.zeros((16,), jnp.float32)

        def zb_body(i, _):
            for h in range(nh):
                rows_v[0, i, pl.ds(h * 16, 16)] = z16
            return 0
        lax.fori_loop(0, 128, zb_body, 0)
        rbase = s * 3200
        for k in range(25):
            pltpu.sync_copy(rows_v.at[0].at[pl.ds(0, 128)],
                            aggr.at[pl.ds(rbase + k * 128, 128)])
        plsc.subcore_barrier()

        def ed_cps(k, q):
            return [
                pltpu.make_async_copy(edata.at[pl.ds(base + k * rps, rps)],
                                      ed_v.at[q], sem_i),
                pltpu.make_async_copy(eattr.at[pl.ds(base + k * rps, rps)],
                                      ea_v.at[q], sem_i),
            ]

        def gather_cps(a, q):
            return [pltpu.make_async_copy(
                table.at[ed_v.at[q].at[r].at[0]],
                rows_v.at[a].at[pl.ds(r * 128, 128)], sem_g)
                for r in range(rps)]

        def scatter_cps(a, q):
            return [pltpu.make_async_copy(
                rows_v.at[a].at[pl.ds(r * 128, 128)],
                aggr.at[ed_v.at[q].at[r].at[1]], sem_s)
                for r in range(rps)]

        def bump(q):
            if not l1:
                for r in range(rps):
                    for k in range(8):
                        ed_v[q, r, 0, pl.ds(k * 16, 16)] = (
                            ed_v[q, r, 0, pl.ds(k * 16, 16)] + cN)

        def compute(a, q):
            def grp(g, _):
                r = g >> 3
                eo = (g & 7) * 16
                ea0g = ea_v[q, r, 0, pl.ds(eo, 16)]
                ea1g = ea_v[q, r, 1, pl.ds(eo, 16)]
                for l in range(16):
                    j = g * 16 + l
                    for h in range(nh):
                        r0 = rows_v[a, j, pl.ds(h * 16, 16)]
                        rows_v[a, j, pl.ds(h * 16, 16)] = jnp.maximum(
                            r0 + ea0g[l] * ws[0][h] + ea1g[l] * ws[1][h]
                            + ws[2][h], 0.0)
                return 0
            lax.fori_loop(0, 8 * rps, grp, 0)

        for cp in ed_cps(0, 0):
            cp.start()

        def hex_body(p, _):
            for u in range(6):
                k = p * 6 + u
                a = u % 2
                for cp in ed_cps(k, u):
                    cp.wait()
                bump(u)
                if u < 5:
                    for cp in ed_cps(k + 1, u + 1):
                        cp.start()
                else:
                    @pl.when(p < HEX - 1)
                    def _(k=k):
                        for cp in ed_cps(k + 1, 0):
                            cp.start()
                if u >= 2:
                    for cp in scatter_cps(a, (u + 4) % 6):
                        cp.wait()
                else:
                    @pl.when(p > 0)
                    def _(u=u, a=a):
                        for cp in scatter_cps(a, (u + 4) % 6):
                            cp.wait()
                for cp in gather_cps(a, u):
                    cp.start()
                if u >= 1:
                    for cp in gather_cps(1 - a, u - 1):
                        cp.wait()
                    compute(1 - a, u - 1)
                    for cp in scatter_cps(1 - a, u - 1):
                        cp.start(add=True)
                else:
                    @pl.when(p > 0)
                    def _():
                        for cp in gather_cps(1, 5):
                            cp.wait()
                        compute(1, 5)
                        for cp in scatter_cps(1, 5):
                            cp.start(add=True)
            return 0
        lax.fori_loop(0, HEX, hex_body, 0)
        for cp in gather_cps(1, 5):
            cp.wait()
        compute(1, 5)
        for cp in scatter_cps(1, 5):
            cp.start(add=True)
        for a, q in ((0, 4), (1, 5)):
            for cp in scatter_cps(a, q):
                cp.wait()
        plsc.subcore_barrier()

        for k in range(5):
            pltpu.sync_copy(aggr.at[pl.ds(rbase + k * 640, 640)],
                            out.at[pl.ds(c * NP + rbase + k * 640, 640)])
    return body


_sc_msg16_body = _make_msg_body(16, True)
_sc_msg32_body = _make_msg_body(32, False)


def _sc_msg16(x16, edata, ea3, wpk):
    return pl.kernel(
        _sc_msg16_body,
        out_type=jax.ShapeDtypeStruct((2 * NP, 16), jnp.float32),
        mesh=_mesh,
        compiler_params=pltpu.CompilerParams(use_tc_tiling_on_sc=False),
        scratch_types=[
            pltpu.VMEM((6, 1, 2, 128), jnp.int32),
            pltpu.VMEM((6, 1, 2, 128), jnp.float32),
            pltpu.VMEM((2, 128, 16), jnp.float32),
            pltpu.VMEM((3, 16), jnp.float32),
            pltpu.VMEM_SHARED((NP, 16), jnp.float32),
            pltpu.SemaphoreType.DMA,
            pltpu.SemaphoreType.DMA,
            pltpu.SemaphoreType.DMA,
        ],
    )(x16, edata, ea3, wpk)


def _sc_msg32(table, edata, ea3, wpk):
    return pl.kernel(
        _sc_msg32_body,
        out_type=jax.ShapeDtypeStruct((2 * NP, 32), jnp.float32),
        mesh=_mesh,
        compiler_params=pltpu.CompilerParams(use_tc_tiling_on_sc=False),
        scratch_types=[
            pltpu.VMEM((6, 2, 2, 128), jnp.int32),
            pltpu.VMEM((6, 2, 2, 128), jnp.float32),
            pltpu.VMEM((2, 256, 32), jnp.float32),
            pltpu.VMEM((3, 32), jnp.float32),
            pltpu.VMEM_SHARED((NP, 32), jnp.float32),
            pltpu.SemaphoreType.DMA,
            pltpu.SemaphoreType.DMA,
            pltpu.SemaphoreType.DMA,
        ],
    )(table, edata, ea3, wpk)


def _dense_l1_body(x_ref, a1_ref, a2_ref, W_ref, b_ref, g_ref, be_ref,
                   Wr_ref, br_ref, out_ref, t_s, st_s):
    i = pl.program_id(0)
    blk = lax.rem(i, NBLK)

    @pl.when(i < NBLK)
    def _():
        ag = a1_ref[0] + a2_ref[0]
        t = jnp.dot(x_ref[...] + ag, W_ref[...],
                    preferred_element_type=jnp.float32) + b_ref[...]
        t_s[pl.ds(blk * BLK, BLK), :] = t
        rowid = lax.broadcasted_iota(jnp.int32, (BLK, 1), 0) + blk * BLK
        tm = t * (rowid < N).astype(jnp.float32)

        @pl.when(i == 0)
        def _():
            st_s[...] = jnp.zeros_like(st_s)
        st_s[0:1, :] += jnp.sum(tm, axis=0, keepdims=True)
        st_s[1:2, :] += jnp.sum(tm * tm, axis=0, keepdims=True)

    @pl.when(i >= NBLK)
    def _():
        t = t_s[pl.ds(blk * BLK, BLK), :]
        hb = _bn_apply(t, st_s[...], g_ref[...], be_ref[...])
        res = jnp.dot(x_ref[...], Wr_ref[...],
                      preferred_element_type=jnp.float32) + br_ref[...]
        h = hb + res
        out_ref[0] = h[:, 0:32]
        out_ref[1] = h[:, 32:64]


def _dense_mid_body(x1_ref, x2_ref, a1_ref, a2_ref, W_ref, b_ref, g_ref,
                    be_ref, out_ref, t_s, st_s):
    i = pl.program_id(0)
    blk = lax.rem(i, NBLK)

    @pl.when(i < NBLK)
    def _():
        xin = jnp.concatenate([x1_ref[0], x2_ref[0]], axis=1)
        ag = jnp.concatenate([a1_ref[0], a2_ref[0]], axis=1)
        t = jnp.dot(xin + ag, W_ref[...],
                    preferred_element_type=jnp.float32) + b_ref[...]
        t_s[pl.ds(blk * BLK, BLK), :] = t
        rowid = lax.broadcasted_iota(jnp.int32, (BLK, 1), 0) + blk * BLK
        tm = t * (rowid < N).astype(jnp.float32)

        @pl.when(i == 0)
        def _():
            st_s[...] = jnp.zeros_like(st_s)
        st_s[0:1, :] += jnp.sum(tm, axis=0, keepdims=True)
        st_s[1:2, :] += jnp.sum(tm * tm, axis=0, keepdims=True)

    @pl.when(i >= NBLK)
    def _():
        t = t_s[pl.ds(blk * BLK, BLK), :]
        hb = _bn_apply(t, st_s[...], g_ref[...], be_ref[...])
        res = jnp.concatenate([x1_ref[0], x2_ref[0]], axis=1)
        h = hb + res
        out_ref[0] = h[:, 0:32]
        out_ref[1] = h[:, 32:64]


def _dense_l3_body(x1_ref, x2_ref, a1_ref, a2_ref, W_ref, b_ref, g_ref,
                   be_ref, out_ref, t_s, st_s):
    i = pl.program_id(0)
    blk = lax.rem(i, NBLK)

    @pl.when(i < NBLK)
    def _():
        xin = jnp.concatenate([x1_ref[0], x2_ref[0]], axis=1)
        ag = jnp.concatenate([a1_ref[0], a2_ref[0]], axis=1)
        t = jnp.dot(xin + ag, W_ref[...],
                    preferred_element_type=jnp.float32) + b_ref[...]
        t_s[pl.ds(blk * BLK, BLK), :] = t
        rowid = lax.broadcasted_iota(jnp.int32, (BLK, 1), 0) + blk * BLK
        tm = t * (rowid < N).astype(jnp.float32)

        @pl.when(i == 0)
        def _():
            st_s[...] = jnp.zeros_like(st_s)
        st_s[0:1, :] += jnp.sum(tm, axis=0, keepdims=True)
        st_s[1:2, :] += jnp.sum(tm * tm, axis=0, keepdims=True)

    @pl.when(i >= NBLK)
    def _():
        t = t_s[pl.ds(blk * BLK, BLK), :]
        hb = _bn_apply(t, st_s[...], g_ref[...], be_ref[...])
        res = jnp.concatenate([x1_ref[0], x2_ref[0]], axis=1)
        out_ref[...] = hb + res


def _bn_apply(t, st, g, be):
    mu = st[0:1, :] * (1.0 / N)
    ex2 = st[1:2, :] * (1.0 / N)
    var = ex2 - mu * mu
    inv = lax.rsqrt(var + BN_EPS)
    return jnp.maximum((t - mu) * inv * g + be, 0.0)


def _dense_l1(x16, p, W, b, g, be, Wr, br):
    return pl.pallas_call(
        _dense_l1_body,
        grid=(2 * NBLK,),
        in_specs=[
            pl.BlockSpec((BLK, 16), lambda i: (i % NBLK, 0)),
            pl.BlockSpec((1, BLK, 16), lambda i: (0, i % NBLK, 0)),
            pl.BlockSpec((1, BLK, 16), lambda i: (1, i % NBLK, 0)),
            pl.BlockSpec((16, 64), lambda i: (0, 0)),
            pl.BlockSpec((1, 64), lambda i: (0, 0)),
            pl.BlockSpec((1, 64), lambda i: (0, 0)),
            pl.BlockSpec((1, 64), lambda i: (0, 0)),
            pl.BlockSpec((16, 64), lambda i: (0, 0)),
            pl.BlockSpec((1, 64), lambda i: (0, 0)),
        ],
        out_specs=pl.BlockSpec((2, BLK, 32), lambda i: (0, i % NBLK, 0)),
        out_shape=jax.ShapeDtypeStruct((2, NP, 32), jnp.float32),
        scratch_shapes=[
            pltpu.VMEM((NP, 64), jnp.float32),
            pltpu.VMEM((8, 64), jnp.float32),
        ],
    )(x16, p, p, W, b, g, be, Wr, br)


def _dense_mid(h, a, W, b, g, be):
    return pl.pallas_call(
        _dense_mid_body,
        grid=(2 * NBLK,),
        in_specs=[
            pl.BlockSpec((1, BLK, 32), lambda i: (0, i % NBLK, 0)),
            pl.BlockSpec((1, BLK, 32), lambda i: (1, i % NBLK, 0)),
            pl.BlockSpec((1, BLK, 32), lambda i: (0, i % NBLK, 0)),
            pl.BlockSpec((1, BLK, 32), lambda i: (1, i % NBLK, 0)),
            pl.BlockSpec((64, 64), lambda i: (0, 0)),
            pl.BlockSpec((1, 64), lambda i: (0, 0)),
            pl.BlockSpec((1, 64), lambda i: (0, 0)),
            pl.BlockSpec((1, 64), lambda i: (0, 0)),
        ],
        out_specs=pl.BlockSpec((2, BLK, 32), lambda i: (0, i % NBLK, 0)),
        out_shape=jax.ShapeDtypeStruct((2, NP, 32), jnp.float32),
        scratch_shapes=[
            pltpu.VMEM((NP, 64), jnp.float32),
            pltpu.VMEM((8, 64), jnp.float32),
        ],
    )(h, h, a, a, W, b, g, be)


def _dense_l3(h, a, W, b, g, be):
    return pl.pallas_call(
        _dense_l3_body,
        grid=(2 * NBLK,),
        in_specs=[
            pl.BlockSpec((1, BLK, 32), lambda i: (0, i % NBLK, 0)),
            pl.BlockSpec((1, BLK, 32), lambda i: (1, i % NBLK, 0)),
            pl.BlockSpec((1, BLK, 32), lambda i: (0, i % NBLK, 0)),
            pl.BlockSpec((1, BLK, 32), lambda i: (1, i % NBLK, 0)),
            pl.BlockSpec((64, 64), lambda i: (0, 0)),
            pl.BlockSpec((1, 64), lambda i: (0, 0)),
            pl.BlockSpec((1, 64), lambda i: (0, 0)),
            pl.BlockSpec((1, 64), lambda i: (0, 0)),
        ],
        out_specs=pl.BlockSpec((BLK, 64), lambda i: (i % NBLK, 0)),
        out_shape=jax.ShapeDtypeStruct((NP, 64), jnp.float32),
        scratch_shapes=[
            pltpu.VMEM((NP, 64), jnp.float32),
            pltpu.VMEM((8, 64), jnp.float32),
        ],
    )(h, h, a, a, W, b, g, be)


def _sc_pool(h, batch_pad):
    def body(h_hbm, b_hbm, ps_o, pm_o, pc_o, hv, bv, ps_v, pm_v, pc_v, sem):
        c = lax.axis_index("c")
        s = lax.axis_index("s")
        wid = s * 2 + c
        z16 = jnp.zeros((16,), jnp.float32)
        ninf = jnp.full((16,), -jnp.inf, jnp.float32)

        e0 = jnp.where(lax.broadcasted_iota(jnp.int32, (16,), 0) == 0, 1.0, 0.0)

        def init(i, _):
            for k in range(4):
                ps_v[i, pl.ds(k * 16, 16)] = z16
                pm_v[i, pl.ds(k * 16, 16)] = ninf
            pc_v[i, pl.ds(0, 16)] = z16
            return 0
        lax.fori_loop(0, GP, init, 0)

        base = wid * 12 + jnp.minimum(wid, 16)
        cnt = 12 + (wid < 16).astype(jnp.int32)

        def chunk(i, _):
            row0 = (base + i) * 128
            pltpu.sync_copy(h_hbm.at[pl.ds(row0, 128)], hv)
            pltpu.sync_copy(b_hbm.at[pl.ds(row0, 128)], bv)

            def rowf(g, _):
                gv = bv[pl.ds(g * 16, 16)]
                for l in range(16):
                    j = g * 16 + l
                    gid = gv[l]
                    for k in range(4):
                        hk = hv[j, pl.ds(k * 16, 16)]
                        ps_v[gid, pl.ds(k * 16, 16)] = ps_v[gid, pl.ds(k * 16, 16)] + hk
                        pm_v[gid, pl.ds(k * 16, 16)] = jnp.maximum(
                            pm_v[gid, pl.ds(k * 16, 16)], hk)
                    pc_v[gid, pl.ds(0, 16)] = pc_v[gid, pl.ds(0, 16)] + e0
                return 0
            lax.fori_loop(0, 8, rowf, 0)
            return 0
        lax.fori_loop(0, cnt, chunk, 0)

        pltpu.sync_copy(ps_v, ps_o.at[wid])
        pltpu.sync_copy(pm_v, pm_o.at[wid])
        pltpu.sync_copy(pc_v, pc_o.at[wid])

    return pl.kernel(
        body,
        out_type=[
            jax.ShapeDtypeStruct((32, GP, 64), jnp.float32),
            jax.ShapeDtypeStruct((32, GP, 64), jnp.float32),
            jax.ShapeDtypeStruct((32, GP, 16), jnp.float32),
        ],
        mesh=_mesh,
        compiler_params=pltpu.CompilerParams(use_tc_tiling_on_sc=False),
        scratch_types=[
            pltpu.VMEM((128, 64), jnp.float32),
            pltpu.VMEM((128,), jnp.int32),
            pltpu.VMEM((GP, 64), jnp.float32),
            pltpu.VMEM((GP, 64), jnp.float32),
            pltpu.VMEM((GP, 16), jnp.float32),
            pltpu.SemaphoreType.DMA,
        ],
    )(h, batch_pad)


def _cls_body(ps_ref, pm_ref, pc_ref, w1_ref, b1_ref, w2_ref, b2_ref, out_ref):
    s = jnp.sum(ps_ref[...][:, 0:G, :], axis=0)
    m = jnp.max(pm_ref[...][:, 0:G, :], axis=0)
    cc = jnp.sum(pc_ref[...][:, 0:G, 0], axis=0)
    mean = s / jnp.maximum(cc, 1.0)[:, None]
    m = jnp.where(jnp.isfinite(m), m, 0.0)
    z = jnp.concatenate([mean, m], axis=1)
    h1 = jnp.maximum(jnp.dot(z, w1_ref[...], preferred_element_type=jnp.float32)
                     + b1_ref[...], 0.0)
    out_ref[...] = jnp.dot(h1, w2_ref[...],
                           preferred_element_type=jnp.float32) + b2_ref[...]


def _classifier(ps, pm, pc, Wk1, bk1, Wk2, bk2):
    return pl.pallas_call(
        _cls_body,
        out_shape=jax.ShapeDtypeStruct((G, 10), jnp.float32),
    )(ps, pm, pc, Wk1, bk1.reshape(1, -1), Wk2, bk2.reshape(1, -1))


def kernel(x, edge_index, edge_attr, batch,
           We1, be1, We2, be2, We3, be3,
           Wc1, bc1, g1, t1, Wc2, bc2, g2, t2, Wc3, bc3, g3, t3,
           Wr, br, Wk1, bk1, Wk2, bk2):
    f32 = jnp.float32
    x16 = jnp.zeros((NP, 16), f32).at[:N, :12].set(x)
    edata = jnp.concatenate([
        edge_index.reshape(2, ER, 128).transpose(1, 0, 2),
        jnp.concatenate([jnp.zeros((86, 1, 128), jnp.int32),
                         jnp.full((86, 1, 128), N, jnp.int32)], axis=1),
    ], axis=0)
    ea3 = jnp.concatenate([
        edge_attr.T.reshape(2, ER, 128).transpose(1, 0, 2),
        jnp.zeros((86, 2, 128), f32),
    ], axis=0)
    batch_pad = jnp.concatenate([batch, jnp.full((NP - N,), G, jnp.int32)])

    w1pk = jnp.zeros((3, 16), f32).at[0:2, 0:12].set(We1).at[2, 0:12].set(be1)
    w2pk = jnp.stack([
        jnp.stack([We2[0, 0:32], We2[1, 0:32], be2[0:32]]),
        jnp.stack([We2[0, 32:64], We2[1, 32:64], be2[32:64]]),
    ])
    w3pk = jnp.stack([
        jnp.stack([We3[0, 0:32], We3[1, 0:32], be3[0:32]]),
        jnp.stack([We3[0, 32:64], We3[1, 32:64], be3[32:64]]),
    ])
    Wc1p = jnp.zeros((16, 64), f32).at[0:12, :].set(Wc1)
    Wrp = jnp.zeros((16, 64), f32).at[0:12, :].set(Wr)

    p1 = _sc_msg16(x16, edata, ea3, w1pk).reshape(2, NP, 16)
    h1 = _dense_l1(x16, p1, Wc1p, bc1.reshape(1, -1), g1.reshape(1, -1),
                   t1.reshape(1, -1), Wrp, br.reshape(1, -1))

    a2 = _sc_msg32(h1.reshape(2 * NP, 32), edata, ea3, w2pk).reshape(2, NP, 32)
    h2 = _dense_mid(h1, a2, Wc2, bc2.reshape(1, -1), g2.reshape(1, -1),
                    t2.reshape(1, -1))

    a3 = _sc_msg32(h2.reshape(2 * NP, 32), edata, ea3, w3pk).reshape(2, NP, 32)
    h3 = _dense_l3(h2, a3, Wc3, bc3.reshape(1, -1), g3.reshape(1, -1),
                   t3.reshape(1, -1))

    ps, pm, pc = _sc_pool(h3, batch_pad)
    return _classifier(ps, pm, pc, Wk1, bk1, Wk2, bk2)


# speedup vs baseline: 1.0179x; 1.0179x over previous
import functools

import jax
import jax.numpy as jnp
from jax import lax
from jax.experimental import pallas as pl
from jax.experimental.pallas import tpu as pltpu
from jax.experimental.pallas import tpu_sc as plsc

N = 50000
NP = 51200
E = 800000
ER = E // 128
G = 256
GP = 272
BN_EPS = 1e-5
BLK = 1024
NBLK = NP // BLK

_mesh = plsc.VectorSubcoreMesh(core_axis_name="c", subcore_axis_name="s")


def _make_msg_body(fw, l1):
    nh = fw // 16
    rps = 1 if l1 else 2
    S = 198
    HEX = S // 6

    def body(table, edata, eattr, wpk, out, ed_v, ea_v, rows_v, w_v, aggr,
             sem_i, sem_g, sem_s):
        c = lax.axis_index("c")
        s = lax.axis_index("s")
        if l1:
            pltpu.sync_copy(wpk, w_v)
            base = (s * 2 + c) * S
            cN = None
        else:
            pltpu.sync_copy(wpk.at[c], w_v)
            base = s * S * 2
            cN = c * NP
        ws = [[w_v[r, pl.ds(h * 16, 16)] for h in range(nh)] for r in range(3)]
        z16 = jnp.zeros((16,), jnp.float32)

        def zb_body(i, _):
            for h in range(nh):
                rows_v[0, i, pl.ds(h * 16, 16)] = z16
            return 0
        lax.fori_loop(0, 128, zb_body, 0)
        rbase = s * 3200
        for k in range(25):
            pltpu.sync_copy(rows_v.at[0].at[pl.ds(0, 128)],
                            aggr.at[pl.ds(rbase + k * 128, 128)])
        plsc.subcore_barrier()

        def ed_cps(k, q):
            return [
                pltpu.make_async_copy(edata.at[pl.ds(base + k * rps, rps)],
                                      ed_v.at[q], sem_i),
                pltpu.make_async_copy(eattr.at[pl.ds(base + k * rps, rps)],
                                      ea_v.at[q], sem_i),
            ]

        def gather_cps(a, q):
            return [pltpu.make_async_copy(
                table.at[ed_v.at[q].at[r].at[0]],
                rows_v.at[a].at[pl.ds(r * 128, 128)], sem_g)
                for r in range(rps)]

        def scatter_cps(a, q):
            return [pltpu.make_async_copy(
                rows_v.at[a].at[pl.ds(r * 128, 128)],
                aggr.at[ed_v.at[q].at[r].at[1]], sem_s)
                for r in range(rps)]

        def bump(q):
            if not l1:
                for r in range(rps):
                    for k in range(8):
                        ed_v[q, r, 0, pl.ds(k * 16, 16)] = (
                            ed_v[q, r, 0, pl.ds(k * 16, 16)] + cN)

        def compute(a, q):
            def grp(g, _):
                r = g >> 3
                eo = (g & 7) * 16
                ea0g = ea_v[q, r, 0, pl.ds(eo, 16)]
                ea1g = ea_v[q, r, 1, pl.ds(eo, 16)]
                for l in range(16):
                    j = g * 16 + l
                    for h in range(nh):
                        r0 = rows_v[a, j, pl.ds(h * 16, 16)]
                        rows_v[a, j, pl.ds(h * 16, 16)] = jnp.maximum(
                            r0 + ea0g[l] * ws[0][h] + ea1g[l] * ws[1][h]
                            + ws[2][h], 0.0)
                return 0
            lax.fori_loop(0, 8 * rps, grp, 0)

        for cp in ed_cps(0, 0):
            cp.start()

        def hex_body(p, _):
            for u in range(6):
                k = p * 6 + u
                a = u % 2
                for cp in ed_cps(k, u):
                    cp.wait()
                bump(u)
                if u < 5:
                    for cp in ed_cps(k + 1, u + 1):
                        cp.start()
                else:
                    @pl.when(p < HEX - 1)
                    def _(k=k):
                        for cp in ed_cps(k + 1, 0):
                            cp.start()
                if u >= 2:
                    for cp in scatter_cps(a, (u + 4) % 6):
                        cp.wait()
                else:
                    @pl.when(p > 0)
                    def _(u=u, a=a):
                        for cp in scatter_cps(a, (u + 4) % 6):
                            cp.wait()
                for cp in gather_cps(a, u):
                    cp.start()
                if u >= 1:
                    for cp in gather_cps(1 - a, u - 1):
                        cp.wait()
                    compute(1 - a, u - 1)
                    for cp in scatter_cps(1 - a, u - 1):
                        cp.start(add=True)
                else:
                    @pl.when(p > 0)
                    def _():
                        for cp in gather_cps(1, 5):
                            cp.wait()
                        compute(1, 5)
                        for cp in scatter_cps(1, 5):
                            cp.start(add=True)
            return 0
        lax.fori_loop(0, HEX, hex_body, 0)
        for cp in gather_cps(1, 5):
            cp.wait()
        compute(1, 5)
        for cp in scatter_cps(1, 5):
            cp.start(add=True)
        for a, q in ((0, 4), (1, 5)):
            for cp in scatter_cps(a, q):
                cp.wait()
        plsc.subcore_barrier()

        for k in range(5):
            pltpu.sync_copy(aggr.at[pl.ds(rbase + k * 640, 640)],
                            out.at[pl.ds(c * NP + rbase + k * 640, 640)])
    return body


_sc_msg16_body = _make_msg_body(16, True)
_sc_msg32_body = _make_msg_body(32, False)


def _sc_msg16(x16, edata, ea3, wpk):
    return pl.kernel(
        _sc_msg16_body,
        out_type=jax.ShapeDtypeStruct((2 * NP, 16), jnp.float32),
        mesh=_mesh,
        compiler_params=pltpu.CompilerParams(use_tc_tiling_on_sc=False),
        scratch_types=[
            pltpu.VMEM((6, 1, 2, 128), jnp.int32),
            pltpu.VMEM((6, 1, 2, 128), jnp.float32),
            pltpu.VMEM((2, 128, 16), jnp.float32),
            pltpu.VMEM((3, 16), jnp.float32),
            pltpu.VMEM_SHARED((NP, 16), jnp.float32),
            pltpu.SemaphoreType.DMA,
            pltpu.SemaphoreType.DMA,
            pltpu.SemaphoreType.DMA,
        ],
    )(x16, edata, ea3, wpk)


def _sc_msg32(table, edata, ea3, wpk):
    return pl.kernel(
        _sc_msg32_body,
        out_type=jax.ShapeDtypeStruct((2 * NP, 32), jnp.float32),
        mesh=_mesh,
        compiler_params=pltpu.CompilerParams(use_tc_tiling_on_sc=False),
        scratch_types=[
            pltpu.VMEM((6, 2, 2, 128), jnp.int32),
            pltpu.VMEM((6, 2, 2, 128), jnp.float32),
            pltpu.VMEM((2, 256, 32), jnp.float32),
            pltpu.VMEM((3, 32), jnp.float32),
            pltpu.VMEM_SHARED((NP, 32), jnp.float32),
            pltpu.SemaphoreType.DMA,
            pltpu.SemaphoreType.DMA,
            pltpu.SemaphoreType.DMA,
        ],
    )(table, edata, ea3, wpk)


def _dense_l1_body(x_ref, a1_ref, a2_ref, W_ref, b_ref, g_ref, be_ref,
                   Wr_ref, br_ref, out_ref, t_s, st_s):
    i = pl.program_id(0)
    blk = lax.rem(i, NBLK)

    @pl.when(i < NBLK)
    def _():
        ag = a1_ref[0] + a2_ref[0]
        t = jnp.dot(x_ref[...] + ag, W_ref[...],
                    preferred_element_type=jnp.float32) + b_ref[...]
        t_s[pl.ds(blk * BLK, BLK), :] = t
        rowid = lax.broadcasted_iota(jnp.int32, (BLK, 1), 0) + blk * BLK
        tm = t * (rowid < N).astype(jnp.float32)

        @pl.when(i == 0)
        def _():
            st_s[...] = jnp.zeros_like(st_s)
        st_s[0:1, :] += jnp.sum(tm, axis=0, keepdims=True)
        st_s[1:2, :] += jnp.sum(tm * tm, axis=0, keepdims=True)

    @pl.when(i >= NBLK)
    def _():
        t = t_s[pl.ds(blk * BLK, BLK), :]
        hb = _bn_apply(t, st_s[...], g_ref[...], be_ref[...])
        res = jnp.dot(x_ref[...], Wr_ref[...],
                      preferred_element_type=jnp.float32) + br_ref[...]
        h = hb + res
        out_ref[0] = h[:, 0:32]
        out_ref[1] = h[:, 32:64]


def _dense_mid_body(x1_ref, x2_ref, a1_ref, a2_ref, W_ref, b_ref, g_ref,
                    be_ref, out_ref, t_s, st_s):
    i = pl.program_id(0)
    blk = lax.rem(i, NBLK)

    @pl.when(i < NBLK)
    def _():
        xin = jnp.concatenate([x1_ref[0], x2_ref[0]], axis=1)
        ag = jnp.concatenate([a1_ref[0], a2_ref[0]], axis=1)
        t = jnp.dot(xin + ag, W_ref[...],
                    preferred_element_type=jnp.float32) + b_ref[...]
        t_s[pl.ds(blk * BLK, BLK), :] = t
        rowid = lax.broadcasted_iota(jnp.int32, (BLK, 1), 0) + blk * BLK
        tm = t * (rowid < N).astype(jnp.float32)

        @pl.when(i == 0)
        def _():
            st_s[...] = jnp.zeros_like(st_s)
        st_s[0:1, :] += jnp.sum(tm, axis=0, keepdims=True)
        st_s[1:2, :] += jnp.sum(tm * tm, axis=0, keepdims=True)

    @pl.when(i >= NBLK)
    def _():
        t = t_s[pl.ds(blk * BLK, BLK), :]
        hb = _bn_apply(t, st_s[...], g_ref[...], be_ref[...])
        res = jnp.concatenate([x1_ref[0], x2_ref[0]], axis=1)
        h = hb + res
        out_ref[0] = h[:, 0:32]
        out_ref[1] = h[:, 32:64]


def _dense_l3_body(x1_ref, x2_ref, a1_ref, a2_ref, W_ref, b_ref, g_ref,
                   be_ref, out_ref, t_s, st_s):
    i = pl.program_id(0)
    blk = lax.rem(i, NBLK)

    @pl.when(i < NBLK)
    def _():
        xin = jnp.concatenate([x1_ref[0], x2_ref[0]], axis=1)
        ag = jnp.concatenate([a1_ref[0], a2_ref[0]], axis=1)
        t = jnp.dot(xin + ag, W_ref[...],
                    preferred_element_type=jnp.float32) + b_ref[...]
        t_s[pl.ds(blk * BLK, BLK), :] = t
        rowid = lax.broadcasted_iota(jnp.int32, (BLK, 1), 0) + blk * BLK
        tm = t * (rowid < N).astype(jnp.float32)

        @pl.when(i == 0)
        def _():
            st_s[...] = jnp.zeros_like(st_s)
        st_s[0:1, :] += jnp.sum(tm, axis=0, keepdims=True)
        st_s[1:2, :] += jnp.sum(tm * tm, axis=0, keepdims=True)

    @pl.when(i >= NBLK)
    def _():
        t = t_s[pl.ds(blk * BLK, BLK), :]
        hb = _bn_apply(t, st_s[...], g_ref[...], be_ref[...])
        res = jnp.concatenate([x1_ref[0], x2_ref[0]], axis=1)
        out_ref[...] = hb + res


def _bn_apply(t, st, g, be):
    mu = st[0:1, :] * (1.0 / N)
    ex2 = st[1:2, :] * (1.0 / N)
    var = ex2 - mu * mu
    inv = lax.rsqrt(var + BN_EPS)
    return jnp.maximum((t - mu) * inv * g + be, 0.0)


def _dense_l1(x16, p, W, b, g, be, Wr, br):
    return pl.pallas_call(
        _dense_l1_body,
        grid=(2 * NBLK,),
        in_specs=[
            pl.BlockSpec((BLK, 16), lambda i: (i % NBLK, 0)),
            pl.BlockSpec((1, BLK, 16), lambda i: (0, i % NBLK, 0)),
            pl.BlockSpec((1, BLK, 16), lambda i: (1, i % NBLK, 0)),
            pl.BlockSpec((16, 64), lambda i: (0, 0)),
            pl.BlockSpec((1, 64), lambda i: (0, 0)),
            pl.BlockSpec((1, 64), lambda i: (0, 0)),
            pl.BlockSpec((1, 64), lambda i: (0, 0)),
            pl.BlockSpec((16, 64), lambda i: (0, 0)),
            pl.BlockSpec((1, 64), lambda i: (0, 0)),
        ],
        out_specs=pl.BlockSpec(
            (2, BLK, 32),
            lambda i: (0, jnp.where(i < NBLK, 0, i % NBLK), 0)),
        out_shape=jax.ShapeDtypeStruct((2, NP, 32), jnp.float32),
        scratch_shapes=[
            pltpu.VMEM((NP, 64), jnp.float32),
            pltpu.VMEM((8, 64), jnp.float32),
        ],
    )(x16, p, p, W, b, g, be, Wr, br)


def _dense_mid(h, a, W, b, g, be):
    return pl.pallas_call(
        _dense_mid_body,
        grid=(2 * NBLK,),
        in_specs=[
            pl.BlockSpec((1, BLK, 32), lambda i: (0, i % NBLK, 0)),
            pl.BlockSpec((1, BLK, 32), lambda i: (1, i % NBLK, 0)),
            pl.BlockSpec((1, BLK, 32), lambda i: (0, i % NBLK, 0)),
            pl.BlockSpec((1, BLK, 32), lambda i: (1, i % NBLK, 0)),
            pl.BlockSpec((64, 64), lambda i: (0, 0)),
            pl.BlockSpec((1, 64), lambda i: (0, 0)),
            pl.BlockSpec((1, 64), lambda i: (0, 0)),
            pl.BlockSpec((1, 64), lambda i: (0, 0)),
        ],
        out_specs=pl.BlockSpec(
            (2, BLK, 32),
            lambda i: (0, jnp.where(i < NBLK, 0, i % NBLK), 0)),
        out_shape=jax.ShapeDtypeStruct((2, NP, 32), jnp.float32),
        scratch_shapes=[
            pltpu.VMEM((NP, 64), jnp.float32),
            pltpu.VMEM((8, 64), jnp.float32),
        ],
    )(h, h, a, a, W, b, g, be)


def _dense_l3(h, a, W, b, g, be):
    return pl.pallas_call(
        _dense_l3_body,
        grid=(2 * NBLK,),
        in_specs=[
            pl.BlockSpec((1, BLK, 32), lambda i: (0, i % NBLK, 0)),
            pl.BlockSpec((1, BLK, 32), lambda i: (1, i % NBLK, 0)),
            pl.BlockSpec((1, BLK, 32), lambda i: (0, i % NBLK, 0)),
            pl.BlockSpec((1, BLK, 32), lambda i: (1, i % NBLK, 0)),
            pl.BlockSpec((64, 64), lambda i: (0, 0)),
            pl.BlockSpec((1, 64), lambda i: (0, 0)),
            pl.BlockSpec((1, 64), lambda i: (0, 0)),
            pl.BlockSpec((1, 64), lambda i: (0, 0)),
        ],
        out_specs=pl.BlockSpec(
            (BLK, 64),
            lambda i: (jnp.where(i < NBLK, 0, i % NBLK), 0)),
        out_shape=jax.ShapeDtypeStruct((NP, 64), jnp.float32),
        scratch_shapes=[
            pltpu.VMEM((NP, 64), jnp.float32),
            pltpu.VMEM((8, 64), jnp.float32),
        ],
    )(h, h, a, a, W, b, g, be)


def _sc_pool(h, batch_pad):
    def body(h_hbm, b_hbm, ps_o, pm_o, pc_o, hv, bv, ps_v, pm_v, pc_v, sem):
        c = lax.axis_index("c")
        s = lax.axis_index("s")
        wid = s * 2 + c
        z16 = jnp.zeros((16,), jnp.float32)
        ninf = jnp.full((16,), -jnp.inf, jnp.float32)

        e0 = jnp.where(lax.broadcasted_iota(jnp.int32, (16,), 0) == 0, 1.0, 0.0)

        def init(i, _):
            for k in range(4):
                ps_v[i, pl.ds(k * 16, 16)] = z16
                pm_v[i, pl.ds(k * 16, 16)] = ninf
            pc_v[i, pl.ds(0, 16)] = z16
            return 0
        lax.fori_loop(0, GP, init, 0)

        base = wid * 12 + jnp.minimum(wid, 16)
        cnt = 12 + (wid < 16).astype(jnp.int32)

        def chunk(i, _):
            row0 = (base + i) * 128
            pltpu.sync_copy(h_hbm.at[pl.ds(row0, 128)], hv)
            pltpu.sync_copy(b_hbm.at[pl.ds(row0, 128)], bv)

            def rowf(g, _):
                gv = bv[pl.ds(g * 16, 16)]
                for l in range(16):
                    j = g * 16 + l
                    gid = gv[l]
                    for k in range(4):
                        hk = hv[j, pl.ds(k * 16, 16)]
                        ps_v[gid, pl.ds(k * 16, 16)] = ps_v[gid, pl.ds(k * 16, 16)] + hk
                        pm_v[gid, pl.ds(k * 16, 16)] = jnp.maximum(
                            pm_v[gid, pl.ds(k * 16, 16)], hk)
                    pc_v[gid, pl.ds(0, 16)] = pc_v[gid, pl.ds(0, 16)] + e0
                return 0
            lax.fori_loop(0, 8, rowf, 0)
            return 0
        lax.fori_loop(0, cnt, chunk, 0)

        pltpu.sync_copy(ps_v, ps_o.at[wid])
        pltpu.sync_copy(pm_v, pm_o.at[wid])
        pltpu.sync_copy(pc_v, pc_o.at[wid])

    return pl.kernel(
        body,
        out_type=[
            jax.ShapeDtypeStruct((32, GP, 64), jnp.float32),
            jax.ShapeDtypeStruct((32, GP, 64), jnp.float32),
            jax.ShapeDtypeStruct((32, GP, 16), jnp.float32),
        ],
        mesh=_mesh,
        compiler_params=pltpu.CompilerParams(use_tc_tiling_on_sc=False),
        scratch_types=[
            pltpu.VMEM((128, 64), jnp.float32),
            pltpu.VMEM((128,), jnp.int32),
            pltpu.VMEM((GP, 64), jnp.float32),
            pltpu.VMEM((GP, 64), jnp.float32),
            pltpu.VMEM((GP, 16), jnp.float32),
            pltpu.SemaphoreType.DMA,
        ],
    )(h, batch_pad)


def _cls_body(ps_ref, pm_ref, pc_ref, w1_ref, b1_ref, w2_ref, b2_ref, out_ref):
    s = jnp.sum(ps_ref[...][:, 0:G, :], axis=0)
    m = jnp.max(pm_ref[...][:, 0:G, :], axis=0)
    cc = jnp.sum(pc_ref[...][:, 0:G, 0], axis=0)
    mean = s / jnp.maximum(cc, 1.0)[:, None]
    m = jnp.where(jnp.isfinite(m), m, 0.0)
    z = jnp.concatenate([mean, m], axis=1)
    h1 = jnp.maximum(jnp.dot(z, w1_ref[...], preferred_element_type=jnp.float32)
                     + b1_ref[...], 0.0)
    out_ref[...] = jnp.dot(h1, w2_ref[...],
                           preferred_element_type=jnp.float32) + b2_ref[...]


def _classifier(ps, pm, pc, Wk1, bk1, Wk2, bk2):
    return pl.pallas_call(
        _cls_body,
        out_shape=jax.ShapeDtypeStruct((G, 10), jnp.float32),
    )(ps, pm, pc, Wk1, bk1.reshape(1, -1), Wk2, bk2.reshape(1, -1))


def kernel(x, edge_index, edge_attr, batch,
           We1, be1, We2, be2, We3, be3,
           Wc1, bc1, g1, t1, Wc2, bc2, g2, t2, Wc3, bc3, g3, t3,
           Wr, br, Wk1, bk1, Wk2, bk2):
    f32 = jnp.float32
    x16 = jnp.zeros((NP, 16), f32).at[:N, :12].set(x)
    edata = jnp.concatenate([
        edge_index.reshape(2, ER, 128).transpose(1, 0, 2),
        jnp.concatenate([jnp.zeros((86, 1, 128), jnp.int32),
                         jnp.full((86, 1, 128), N, jnp.int32)], axis=1),
    ], axis=0)
    ea3 = jnp.concatenate([
        edge_attr.T.reshape(2, ER, 128).transpose(1, 0, 2),
        jnp.zeros((86, 2, 128), f32),
    ], axis=0)
    batch_pad = jnp.concatenate([batch, jnp.full((NP - N,), G, jnp.int32)])

    w1pk = jnp.zeros((3, 16), f32).at[0:2, 0:12].set(We1).at[2, 0:12].set(be1)
    w2pk = jnp.stack([
        jnp.stack([We2[0, 0:32], We2[1, 0:32], be2[0:32]]),
        jnp.stack([We2[0, 32:64], We2[1, 32:64], be2[32:64]]),
    ])
    w3pk = jnp.stack([
        jnp.stack([We3[0, 0:32], We3[1, 0:32], be3[0:32]]),
        jnp.stack([We3[0, 32:64], We3[1, 32:64], be3[32:64]]),
    ])
    Wc1p = jnp.zeros((16, 64), f32).at[0:12, :].set(Wc1)
    Wrp = jnp.zeros((16, 64), f32).at[0:12, :].set(Wr)

    p1 = _sc_msg16(x16, edata, ea3, w1pk).reshape(2, NP, 16)
    h1 = _dense_l1(x16, p1, Wc1p, bc1.reshape(1, -1), g1.reshape(1, -1),
                   t1.reshape(1, -1), Wrp, br.reshape(1, -1))

    a2 = _sc_msg32(h1.reshape(2 * NP, 32), edata, ea3, w2pk).reshape(2, NP, 32)
    h2 = _dense_mid(h1, a2, Wc2, bc2.reshape(1, -1), g2.reshape(1, -1),
                    t2.reshape(1, -1))

    a3 = _sc_msg32(h2.reshape(2 * NP, 32), edata, ea3, w3pk).reshape(2, NP, 32)
    h3 = _dense_l3(h2, a3, Wc3, bc3.reshape(1, -1), g3.reshape(1, -1),
                   t3.reshape(1, -1))

    ps, pm, pc = _sc_pool(h3, batch_pad)
    return _classifier(ps, pm, pc, Wk1, bk1, Wk2, bk2)


# speedup vs baseline: 1.1483x; 1.1282x over previous
import functools

import jax
import jax.numpy as jnp
from jax import lax
from jax.experimental import pallas as pl
from jax.experimental.pallas import tpu as pltpu
from jax.experimental.pallas import tpu_sc as plsc

N = 50000
NP = 51200
E = 800000
ER = E // 128
G = 256
GP = 272
BN_EPS = 1e-5
BLK = 1024
NBLK = NP // BLK

_mesh = plsc.VectorSubcoreMesh(core_axis_name="c", subcore_axis_name="s")


def _make_msg_body(fw, l1):
    nh = fw // 16
    rps = 1 if l1 else 2
    S = 198
    HEX = S // 6

    def body(table, edata, eattr, wpk, out, ed_v, ea_v, rows_v, w_v, aggr,
             sem_i, sem_g, sem_s):
        c = lax.axis_index("c")
        s = lax.axis_index("s")
        if l1:
            pltpu.sync_copy(wpk, w_v)
            base = (s * 2 + c) * S
            cN = None
        else:
            pltpu.sync_copy(wpk.at[c], w_v)
            base = s * S * 2
            cN = c * NP
        ws = [[w_v[r, pl.ds(h * 16, 16)] for h in range(nh)] for r in range(3)]
        z16 = jnp.zeros((16,), jnp.float32)

        def zb_body(i, _):
            for h in range(nh):
                rows_v[0, i, pl.ds(h * 16, 16)] = z16
            return 0
        lax.fori_loop(0, 128, zb_body, 0)
        rbase = s * 3200
        for k in range(25):
            pltpu.sync_copy(rows_v.at[0].at[pl.ds(0, 128)],
                            aggr.at[pl.ds(rbase + k * 128, 128)])
        plsc.subcore_barrier()

        def ed_cps(k, q):
            return [
                pltpu.make_async_copy(edata.at[pl.ds(base + k * rps, rps)],
                                      ed_v.at[q], sem_i),
                pltpu.make_async_copy(eattr.at[pl.ds(base + k * rps, rps)],
                                      ea_v.at[q], sem_i),
            ]

        def gather_cps(a, q):
            return [pltpu.make_async_copy(
                table.at[ed_v.at[q].at[r].at[0]],
                rows_v.at[a].at[pl.ds(r * 128, 128)], sem_g)
                for r in range(rps)]

        def scatter_cps(a, q):
            return [pltpu.make_async_copy(
                rows_v.at[a].at[pl.ds(r * 128, 128)],
                aggr.at[ed_v.at[q].at[r].at[1]], sem_s)
                for r in range(rps)]

        def bump(q):
            if not l1:
                for r in range(rps):
                    for k in range(8):
                        ed_v[q, r, 0, pl.ds(k * 16, 16)] = (
                            ed_v[q, r, 0, pl.ds(k * 16, 16)] + cN)

        def compute(a, q):
            @functools.partial(plsc.parallel_loop, 0, 8 * rps, unroll=2)
            def grp(g):
                r = g >> 3
                eo = (g & 7) * 16
                ea0g = ea_v[q, r, 0, pl.ds(eo, 16)]
                ea1g = ea_v[q, r, 1, pl.ds(eo, 16)]
                for l in range(16):
                    j = g * 16 + l
                    for h in range(nh):
                        r0 = rows_v[a, j, pl.ds(h * 16, 16)]
                        rows_v[a, j, pl.ds(h * 16, 16)] = jnp.maximum(
                            r0 + ea0g[l] * ws[0][h] + ea1g[l] * ws[1][h]
                            + ws[2][h], 0.0)

        for cp in ed_cps(0, 0):
            cp.start()

        def hex_body(p, _):
            for u in range(6):
                k = p * 6 + u
                a = u % 2
                for cp in ed_cps(k, u):
                    cp.wait()
                bump(u)
                if u < 5:
                    for cp in ed_cps(k + 1, u + 1):
                        cp.start()
                else:
                    @pl.when(p < HEX - 1)
                    def _(k=k):
                        for cp in ed_cps(k + 1, 0):
                            cp.start()
                if u >= 2:
                    for cp in scatter_cps(a, (u + 4) % 6):
                        cp.wait()
                else:
                    @pl.when(p > 0)
                    def _(u=u, a=a):
                        for cp in scatter_cps(a, (u + 4) % 6):
                            cp.wait()
                for cp in gather_cps(a, u):
                    cp.start()
                if u >= 1:
                    for cp in gather_cps(1 - a, u - 1):
                        cp.wait()
                    compute(1 - a, u - 1)
                    for cp in scatter_cps(1 - a, u - 1):
                        cp.start(add=True)
                else:
                    @pl.when(p > 0)
                    def _():
                        for cp in gather_cps(1, 5):
                            cp.wait()
                        compute(1, 5)
                        for cp in scatter_cps(1, 5):
                            cp.start(add=True)
            return 0
        lax.fori_loop(0, HEX, hex_body, 0)
        for cp in gather_cps(1, 5):
            cp.wait()
        compute(1, 5)
        for cp in scatter_cps(1, 5):
            cp.start(add=True)
        for a, q in ((0, 4), (1, 5)):
            for cp in scatter_cps(a, q):
                cp.wait()
        plsc.subcore_barrier()

        for k in range(5):
            pltpu.sync_copy(aggr.at[pl.ds(rbase + k * 640, 640)],
                            out.at[pl.ds(c * NP + rbase + k * 640, 640)])
    return body


_sc_msg16_body = _make_msg_body(16, True)
_sc_msg32_body = _make_msg_body(32, False)


def _sc_msg16(x16, edata, ea3, wpk):
    return pl.kernel(
        _sc_msg16_body,
        out_type=jax.ShapeDtypeStruct((2 * NP, 16), jnp.float32),
        mesh=_mesh,
        compiler_params=pltpu.CompilerParams(use_tc_tiling_on_sc=False),
        scratch_types=[
            pltpu.VMEM((6, 1, 2, 128), jnp.int32),
            pltpu.VMEM((6, 1, 2, 128), jnp.float32),
            pltpu.VMEM((2, 128, 16), jnp.float32),
            pltpu.VMEM((3, 16), jnp.float32),
            pltpu.VMEM_SHARED((NP, 16), jnp.float32),
            pltpu.SemaphoreType.DMA,
            pltpu.SemaphoreType.DMA,
            pltpu.SemaphoreType.DMA,
        ],
    )(x16, edata, ea3, wpk)


def _sc_msg32(table, edata, ea3, wpk):
    return pl.kernel(
        _sc_msg32_body,
        out_type=jax.ShapeDtypeStruct((2 * NP, 32), jnp.float32),
        mesh=_mesh,
        compiler_params=pltpu.CompilerParams(use_tc_tiling_on_sc=False),
        scratch_types=[
            pltpu.VMEM((6, 2, 2, 128), jnp.int32),
            pltpu.VMEM((6, 2, 2, 128), jnp.float32),
            pltpu.VMEM((2, 256, 32), jnp.float32),
            pltpu.VMEM((3, 32), jnp.float32),
            pltpu.VMEM_SHARED((NP, 32), jnp.float32),
            pltpu.SemaphoreType.DMA,
            pltpu.SemaphoreType.DMA,
            pltpu.SemaphoreType.DMA,
        ],
    )(table, edata, ea3, wpk)


def _dense_l1_body(x_ref, a1_ref, a2_ref, W_ref, b_ref, g_ref, be_ref,
                   Wr_ref, br_ref, out_ref, t_s, st_s):
    i = pl.program_id(0)
    blk = lax.rem(i, NBLK)

    @pl.when(i < NBLK)
    def _():
        ag = a1_ref[0] + a2_ref[0]
        t = jnp.dot(x_ref[...] + ag, W_ref[...],
                    preferred_element_type=jnp.float32) + b_ref[...]
        t_s[pl.ds(blk * BLK, BLK), :] = t
        rowid = lax.broadcasted_iota(jnp.int32, (BLK, 1), 0) + blk * BLK
        tm = t * (rowid < N).astype(jnp.float32)

        @pl.when(i == 0)
        def _():
            st_s[...] = jnp.zeros_like(st_s)
        st_s[0:1, :] += jnp.sum(tm, axis=0, keepdims=True)
        st_s[1:2, :] += jnp.sum(tm * tm, axis=0, keepdims=True)

    @pl.when(i >= NBLK)
    def _():
        t = t_s[pl.ds(blk * BLK, BLK), :]
        hb = _bn_apply(t, st_s[...], g_ref[...], be_ref[...])
        res = jnp.dot(x_ref[...], Wr_ref[...],
                      preferred_element_type=jnp.float32) + br_ref[...]
        h = hb + res
        out_ref[0] = h[:, 0:32]
        out_ref[1] = h[:, 32:64]


def _dense_mid_body(x1_ref, x2_ref, a1_ref, a2_ref, W_ref, b_ref, g_ref,
                    be_ref, out_ref, t_s, st_s):
    i = pl.program_id(0)
    blk = lax.rem(i, NBLK)

    @pl.when(i < NBLK)
    def _():
        xin = jnp.concatenate([x1_ref[0], x2_ref[0]], axis=1)
        ag = jnp.concatenate([a1_ref[0], a2_ref[0]], axis=1)
        t = jnp.dot(xin + ag, W_ref[...],
                    preferred_element_type=jnp.float32) + b_ref[...]
        t_s[pl.ds(blk * BLK, BLK), :] = t
        rowid = lax.broadcasted_iota(jnp.int32, (BLK, 1), 0) + blk * BLK
        tm = t * (rowid < N).astype(jnp.float32)

        @pl.when(i == 0)
        def _():
            st_s[...] = jnp.zeros_like(st_s)
        st_s[0:1, :] += jnp.sum(tm, axis=0, keepdims=True)
        st_s[1:2, :] += jnp.sum(tm * tm, axis=0, keepdims=True)

    @pl.when(i >= NBLK)
    def _():
        t = t_s[pl.ds(blk * BLK, BLK), :]
        hb = _bn_apply(t, st_s[...], g_ref[...], be_ref[...])
        res = jnp.concatenate([x1_ref[0], x2_ref[0]], axis=1)
        h = hb + res
        out_ref[0] = h[:, 0:32]
        out_ref[1] = h[:, 32:64]


def _dense_l3_body(x1_ref, x2_ref, a1_ref, a2_ref, W_ref, b_ref, g_ref,
                   be_ref, out_ref, t_s, st_s):
    i = pl.program_id(0)
    blk = lax.rem(i, NBLK)

    @pl.when(i < NBLK)
    def _():
        xin = jnp.concatenate([x1_ref[0], x2_ref[0]], axis=1)
        ag = jnp.concatenate([a1_ref[0], a2_ref[0]], axis=1)
        t = jnp.dot(xin + ag, W_ref[...],
                    preferred_element_type=jnp.float32) + b_ref[...]
        t_s[pl.ds(blk * BLK, BLK), :] = t
        rowid = lax.broadcasted_iota(jnp.int32, (BLK, 1), 0) + blk * BLK
        tm = t * (rowid < N).astype(jnp.float32)

        @pl.when(i == 0)
        def _():
            st_s[...] = jnp.zeros_like(st_s)
        st_s[0:1, :] += jnp.sum(tm, axis=0, keepdims=True)
        st_s[1:2, :] += jnp.sum(tm * tm, axis=0, keepdims=True)

    @pl.when(i >= NBLK)
    def _():
        t = t_s[pl.ds(blk * BLK, BLK), :]
        hb = _bn_apply(t, st_s[...], g_ref[...], be_ref[...])
        res = jnp.concatenate([x1_ref[0], x2_ref[0]], axis=1)
        out_ref[...] = hb + res


def _bn_apply(t, st, g, be):
    mu = st[0:1, :] * (1.0 / N)
    ex2 = st[1:2, :] * (1.0 / N)
    var = ex2 - mu * mu
    inv = lax.rsqrt(var + BN_EPS)
    return jnp.maximum((t - mu) * inv * g + be, 0.0)


def _dense_l1(x16, p, W, b, g, be, Wr, br):
    return pl.pallas_call(
        _dense_l1_body,
        grid=(2 * NBLK,),
        in_specs=[
            pl.BlockSpec((BLK, 16), lambda i: (i % NBLK, 0)),
            pl.BlockSpec((1, BLK, 16), lambda i: (0, i % NBLK, 0)),
            pl.BlockSpec((1, BLK, 16), lambda i: (1, i % NBLK, 0)),
            pl.BlockSpec((16, 64), lambda i: (0, 0)),
            pl.BlockSpec((1, 64), lambda i: (0, 0)),
            pl.BlockSpec((1, 64), lambda i: (0, 0)),
            pl.BlockSpec((1, 64), lambda i: (0, 0)),
            pl.BlockSpec((16, 64), lambda i: (0, 0)),
            pl.BlockSpec((1, 64), lambda i: (0, 0)),
        ],
        out_specs=pl.BlockSpec(
            (2, BLK, 32),
            lambda i: (0, jnp.where(i < NBLK, 0, i % NBLK), 0)),
        out_shape=jax.ShapeDtypeStruct((2, NP, 32), jnp.float32),
        scratch_shapes=[
            pltpu.VMEM((NP, 64), jnp.float32),
            pltpu.VMEM((8, 64), jnp.float32),
        ],
    )(x16, p, p, W, b, g, be, Wr, br)


def _dense_mid(h, a, W, b, g, be):
    return pl.pallas_call(
        _dense_mid_body,
        grid=(2 * NBLK,),
        in_specs=[
            pl.BlockSpec((1, BLK, 32), lambda i: (0, i % NBLK, 0)),
            pl.BlockSpec((1, BLK, 32), lambda i: (1, i % NBLK, 0)),
            pl.BlockSpec((1, BLK, 32), lambda i: (0, i % NBLK, 0)),
            pl.BlockSpec((1, BLK, 32), lambda i: (1, i % NBLK, 0)),
            pl.BlockSpec((64, 64), lambda i: (0, 0)),
            pl.BlockSpec((1, 64), lambda i: (0, 0)),
            pl.BlockSpec((1, 64), lambda i: (0, 0)),
            pl.BlockSpec((1, 64), lambda i: (0, 0)),
        ],
        out_specs=pl.BlockSpec(
            (2, BLK, 32),
            lambda i: (0, jnp.where(i < NBLK, 0, i % NBLK), 0)),
        out_shape=jax.ShapeDtypeStruct((2, NP, 32), jnp.float32),
        scratch_shapes=[
            pltpu.VMEM((NP, 64), jnp.float32),
            pltpu.VMEM((8, 64), jnp.float32),
        ],
    )(h, h, a, a, W, b, g, be)


def _dense_l3(h, a, W, b, g, be):
    return pl.pallas_call(
        _dense_l3_body,
        grid=(2 * NBLK,),
        in_specs=[
            pl.BlockSpec((1, BLK, 32), lambda i: (0, i % NBLK, 0)),
            pl.BlockSpec((1, BLK, 32), lambda i: (1, i % NBLK, 0)),
            pl.BlockSpec((1, BLK, 32), lambda i: (0, i % NBLK, 0)),
            pl.BlockSpec((1, BLK, 32), lambda i: (1, i % NBLK, 0)),
            pl.BlockSpec((64, 64), lambda i: (0, 0)),
            pl.BlockSpec((1, 64), lambda i: (0, 0)),
            pl.BlockSpec((1, 64), lambda i: (0, 0)),
            pl.BlockSpec((1, 64), lambda i: (0, 0)),
        ],
        out_specs=pl.BlockSpec(
            (BLK, 64),
            lambda i: (jnp.where(i < NBLK, 0, i % NBLK), 0)),
        out_shape=jax.ShapeDtypeStruct((NP, 64), jnp.float32),
        scratch_shapes=[
            pltpu.VMEM((NP, 64), jnp.float32),
            pltpu.VMEM((8, 64), jnp.float32),
        ],
    )(h, h, a, a, W, b, g, be)


def _sc_pool(h, batch_pad):
    def body(h_hbm, b_hbm, ps_o, pm_o, pc_o, hv, bv, ps_v, pm_v, pc_v, sem):
        c = lax.axis_index("c")
        s = lax.axis_index("s")
        wid = s * 2 + c
        z16 = jnp.zeros((16,), jnp.float32)
        ninf = jnp.full((16,), -jnp.inf, jnp.float32)

        e0 = jnp.where(lax.broadcasted_iota(jnp.int32, (16,), 0) == 0, 1.0, 0.0)

        def init(i, _):
            for k in range(4):
                ps_v[i, pl.ds(k * 16, 16)] = z16
                pm_v[i, pl.ds(k * 16, 16)] = ninf
            pc_v[i, pl.ds(0, 16)] = z16
            return 0
        lax.fori_loop(0, GP, init, 0)

        base = wid * 12 + jnp.minimum(wid, 16)
        cnt = 12 + (wid < 16).astype(jnp.int32)

        def chunk(i, _):
            row0 = (base + i) * 128
            pltpu.sync_copy(h_hbm.at[pl.ds(row0, 128)], hv)
            pltpu.sync_copy(b_hbm.at[pl.ds(row0, 128)], bv)

            def rowf(g, _):
                gv = bv[pl.ds(g * 16, 16)]
                for l in range(16):
                    j = g * 16 + l
                    gid = gv[l]
                    for k in range(4):
                        hk = hv[j, pl.ds(k * 16, 16)]
                        ps_v[gid, pl.ds(k * 16, 16)] = ps_v[gid, pl.ds(k * 16, 16)] + hk
                        pm_v[gid, pl.ds(k * 16, 16)] = jnp.maximum(
                            pm_v[gid, pl.ds(k * 16, 16)], hk)
                    pc_v[gid, pl.ds(0, 16)] = pc_v[gid, pl.ds(0, 16)] + e0
                return 0
            lax.fori_loop(0, 8, rowf, 0)
            return 0
        lax.fori_loop(0, cnt, chunk, 0)

        pltpu.sync_copy(ps_v, ps_o.at[wid])
        pltpu.sync_copy(pm_v, pm_o.at[wid])
        pltpu.sync_copy(pc_v, pc_o.at[wid])

    return pl.kernel(
        body,
        out_type=[
            jax.ShapeDtypeStruct((32, GP, 64), jnp.float32),
            jax.ShapeDtypeStruct((32, GP, 64), jnp.float32),
            jax.ShapeDtypeStruct((32, GP, 16), jnp.float32),
        ],
        mesh=_mesh,
        compiler_params=pltpu.CompilerParams(use_tc_tiling_on_sc=False),
        scratch_types=[
            pltpu.VMEM((128, 64), jnp.float32),
            pltpu.VMEM((128,), jnp.int32),
            pltpu.VMEM((GP, 64), jnp.float32),
            pltpu.VMEM((GP, 64), jnp.float32),
            pltpu.VMEM((GP, 16), jnp.float32),
            pltpu.SemaphoreType.DMA,
        ],
    )(h, batch_pad)


def _cls_body(ps_ref, pm_ref, pc_ref, w1_ref, b1_ref, w2_ref, b2_ref, out_ref):
    s = jnp.sum(ps_ref[...][:, 0:G, :], axis=0)
    m = jnp.max(pm_ref[...][:, 0:G, :], axis=0)
    cc = jnp.sum(pc_ref[...][:, 0:G, 0], axis=0)
    mean = s / jnp.maximum(cc, 1.0)[:, None]
    m = jnp.where(jnp.isfinite(m), m, 0.0)
    z = jnp.concatenate([mean, m], axis=1)
    h1 = jnp.maximum(jnp.dot(z, w1_ref[...], preferred_element_type=jnp.float32)
                     + b1_ref[...], 0.0)
    out_ref[...] = jnp.dot(h1, w2_ref[...],
                           preferred_element_type=jnp.float32) + b2_ref[...]


def _classifier(ps, pm, pc, Wk1, bk1, Wk2, bk2):
    return pl.pallas_call(
        _cls_body,
        out_shape=jax.ShapeDtypeStruct((G, 10), jnp.float32),
    )(ps, pm, pc, Wk1, bk1.reshape(1, -1), Wk2, bk2.reshape(1, -1))


def kernel(x, edge_index, edge_attr, batch,
           We1, be1, We2, be2, We3, be3,
           Wc1, bc1, g1, t1, Wc2, bc2, g2, t2, Wc3, bc3, g3, t3,
           Wr, br, Wk1, bk1, Wk2, bk2):
    f32 = jnp.float32
    x16 = jnp.zeros((NP, 16), f32).at[:N, :12].set(x)
    edata = jnp.concatenate([
        edge_index.reshape(2, ER, 128).transpose(1, 0, 2),
        jnp.concatenate([jnp.zeros((86, 1, 128), jnp.int32),
                         jnp.full((86, 1, 128), N, jnp.int32)], axis=1),
    ], axis=0)
    ea3 = jnp.concatenate([
        edge_attr.T.reshape(2, ER, 128).transpose(1, 0, 2),
        jnp.zeros((86, 2, 128), f32),
    ], axis=0)
    batch_pad = jnp.concatenate([batch, jnp.full((NP - N,), G, jnp.int32)])

    w1pk = jnp.zeros((3, 16), f32).at[0:2, 0:12].set(We1).at[2, 0:12].set(be1)
    w2pk = jnp.stack([
        jnp.stack([We2[0, 0:32], We2[1, 0:32], be2[0:32]]),
        jnp.stack([We2[0, 32:64], We2[1, 32:64], be2[32:64]]),
    ])
    w3pk = jnp.stack([
        jnp.stack([We3[0, 0:32], We3[1, 0:32], be3[0:32]]),
        jnp.stack([We3[0, 32:64], We3[1, 32:64], be3[32:64]]),
    ])
    Wc1p = jnp.zeros((16, 64), f32).at[0:12, :].set(Wc1)
    Wrp = jnp.zeros((16, 64), f32).at[0:12, :].set(Wr)

    p1 = _sc_msg16(x16, edata, ea3, w1pk).reshape(2, NP, 16)
    h1 = _dense_l1(x16, p1, Wc1p, bc1.reshape(1, -1), g1.reshape(1, -1),
                   t1.reshape(1, -1), Wrp, br.reshape(1, -1))

    a2 = _sc_msg32(h1.reshape(2 * NP, 32), edata, ea3, w2pk).reshape(2, NP, 32)
    h2 = _dense_mid(h1, a2, Wc2, bc2.reshape(1, -1), g2.reshape(1, -1),
                    t2.reshape(1, -1))

    a3 = _sc_msg32(h2.reshape(2 * NP, 32), edata, ea3, w3pk).reshape(2, NP, 32)
    h3 = _dense_l3(h2, a3, Wc3, bc3.reshape(1, -1), g3.reshape(1, -1),
                   t3.reshape(1, -1))

    ps, pm, pc = _sc_pool(h3, batch_pad)
    return _classifier(ps, pm, pc, Wk1, bk1, Wk2, bk2)
